# Initial kernel scaffold; baseline (speedup 1.0000x reference)
import jax
import jax.numpy as jnp
from jax.experimental import pallas as pl


def kernel(x, edge_index, W1, b1, W2, b2, W3, b3):
    raise NotImplementedError("write your pallas kernel here")


# speedup vs baseline: 12.8913x; 12.8913x over previous
import functools

import jax
import jax.numpy as jnp
from jax import lax
from jax.experimental import pallas as pl
from jax.experimental.pallas import tpu as pltpu
from jax.experimental.pallas import tpu_sc as plsc

N = 10000
E = 320000
NC = 2
NS = 16
K = 80
SLABS = 5
RT = 624
RT_LAST = N - 15 * RT
BR = 1000
NBR = N // BR

_MESH = dict(core_axis_name="c", subcore_axis_name="s")


def _tile_rows(s, copy_fn):
    @pl.when(s < 15)
    def _():
        copy_fn(s * RT, RT)

    @pl.when(s == 15)
    def _():
        copy_fn(15 * RT, RT_LAST)


def _deg_body(edges4d, deg_out, dstbig, ones_v, zb, deg_sh):
    c = lax.axis_index("c")
    s = lax.axis_index("s")
    zero16 = jnp.zeros((16,), jnp.float32)
    for j in range(40):
        zb[pl.ds(16 * j, 16)] = zero16
    one16 = jnp.full((16,), 1.0, jnp.float32)
    for j in range(K // 16):
        ones_v[pl.ds(16 * j, 16)] = one16
    @pl.when(s < 15)
    def _():
        pltpu.sync_copy(zb, deg_sh.at[pl.ds(s * 640, 640)])

    @pl.when(s == 15)
    def _():
        pltpu.sync_copy(zb.at[pl.ds(0, 400)], deg_sh.at[pl.ds(9600, 400)])

    plsc.subcore_barrier()

    @pl.when(c == 0)
    def _():
        ncs = E // NS // K // SLABS

        def slab(r, carry):
            pltpu.sync_copy(edges4d.at[1, s, r], dstbig)

            def it(i, c2):
                pltpu.sync_copy(ones_v, deg_sh.at[dstbig.at[i]], add=True)
                return c2

            return lax.fori_loop(0, ncs, it, carry)

        lax.fori_loop(0, SLABS, slab, 0)

    plsc.subcore_barrier()

    @pl.when(c == 0)
    def _():
        @pl.when(s < 15)
        def _():
            pltpu.sync_copy(deg_sh.at[pl.ds(s * 640, 640)], zb)
            pltpu.sync_copy(zb, deg_out.at[pl.ds(s * 640, 640)])

        @pl.when(s == 15)
        def _():
            pltpu.sync_copy(deg_sh.at[pl.ds(9600, 400)], zb.at[pl.ds(0, 400)])
            pltpu.sync_copy(zb.at[pl.ds(0, 400)], deg_out.at[pl.ds(9600, 400)])


def _deg_call(edges4d):
    ncs = E // NS // K // SLABS
    f = functools.partial(
        pl.kernel,
        out_type=jax.ShapeDtypeStruct((N,), jnp.float32),
        mesh=plsc.VectorSubcoreMesh(**_MESH),
        scratch_types=[
            pltpu.VMEM((ncs, K), jnp.int32),
            pltpu.VMEM((K,), jnp.float32),
            pltpu.VMEM((640,), jnp.float32),
            pltpu.VMEM_SHARED((N,), jnp.float32),
        ],
    )(_deg_body)
    return f(edges4d)


def _make_prop_body(edge_split):
    def body(g_hbm, edges4d, out_hbm, srcbig, dstbig, rows_v, sem, acc_sh):
        c = lax.axis_index("c")
        s = lax.axis_index("s")
        _tile_rows(s, lambda rb, nr: pltpu.sync_copy(
            g_hbm.at[pl.ds(c * N + rb, nr)], acc_sh.at[pl.ds(rb, nr)]))
        if edge_split:
            ncs = E // NC // NS // K // SLABS
            t = c * NS + s
        else:
            ncs = E // NS // K // SLABS
            t = s
        plsc.subcore_barrier()

        def slab(r, carry):
            pltpu.sync_copy(edges4d.at[0, t, r], srcbig)
            pltpu.sync_copy(edges4d.at[1, t, r], dstbig)
            if not edge_split:
                bias = c * N

                def pre(i, c2):
                    for j in range(K // 16):
                        srcbig[i, pl.ds(16 * j, 16)] = (
                            srcbig[i, pl.ds(16 * j, 16)] + bias)
                    return c2

                lax.fori_loop(0, ncs, pre, 0)

            def it(i, c2):
                pltpu.async_copy(g_hbm.at[srcbig.at[i]], rows_v, sem).wait()
                pltpu.sync_copy(rows_v, acc_sh.at[dstbig.at[i]], add=True)
                return c2

            return lax.fori_loop(0, ncs, it, carry)

        lax.fori_loop(0, SLABS, slab, 0)
        plsc.subcore_barrier()
        _tile_rows(s, lambda rb, nr: pltpu.sync_copy(
            acc_sh.at[pl.ds(rb, nr)], out_hbm.at[pl.ds(c * N + rb, nr)]))

    return body


def _prop_call(g_flat, edges4d, edge_split):
    ncs = (E // NC if edge_split else E) // NS // K // SLABS
    f = functools.partial(
        pl.kernel,
        out_type=jax.ShapeDtypeStruct((2 * N, 128), jnp.float32),
        mesh=plsc.VectorSubcoreMesh(**_MESH),
        scratch_types=[
            pltpu.VMEM((ncs, K), jnp.int32),
            pltpu.VMEM((ncs, K), jnp.int32),
            pltpu.VMEM((K, 128), jnp.float32),
            pltpu.SemaphoreType.DMA,
            pltpu.VMEM_SHARED((N, 128), jnp.float32),
        ],
    )(_make_prop_body(edge_split))
    return f(g_flat, edges4d)


def _tc1_body(d_ref, x_ref, w_ref, g_ref):
    dis = lax.rsqrt(d_ref[...] + 1.0)
    m = jnp.dot(x_ref[...], w_ref[...],
                preferred_element_type=jnp.float32)
    g = dis * m
    g_ref[0] = g[:, :128]
    g_ref[1] = g[:, 128:]


def _tc1_call(dcol, x, W1):
    return pl.pallas_call(
        _tc1_body,
        grid=(NBR,),
        in_specs=[
            pl.BlockSpec((BR, 1), lambda r: (r, 0)),
            pl.BlockSpec((BR, 128), lambda r: (r, 0)),
            pl.BlockSpec((128, 256), lambda r: (0, 0)),
        ],
        out_specs=pl.BlockSpec((2, BR, 128), lambda r: (0, r, 0)),
        out_shape=jax.ShapeDtypeStruct((2, N, 128), jnp.float32),
    )(dcol, x, W1)


def _tcmid_body(d_ref, s_ref, b_ref, w_ref, g_ref):
    dis = lax.rsqrt(d_ref[...] + 1.0)
    sfull = jnp.concatenate([s_ref[0], s_ref[1]], axis=1)
    h = jnp.maximum(dis * sfull + b_ref[...], 0.0)
    m = jnp.dot(h, w_ref[...], preferred_element_type=jnp.float32)
    g = dis * m
    g_ref[0] = g[:, :128]
    g_ref[1] = g[:, 128:]


def _tcmid_call(dcol, s3d, b, W):
    return pl.pallas_call(
        _tcmid_body,
        grid=(NBR,),
        in_specs=[
            pl.BlockSpec((BR, 1), lambda r: (r, 0)),
            pl.BlockSpec((2, BR, 128), lambda r: (0, r, 0)),
            pl.BlockSpec((1, 256), lambda r: (0, 0)),
            pl.BlockSpec((256, 256), lambda r: (0, 0)),
        ],
        out_specs=pl.BlockSpec((2, BR, 128), lambda r: (0, r, 0)),
        out_shape=jax.ShapeDtypeStruct((2, N, 128), jnp.float32),
    )(dcol, s3d, b, W)


def _tc3_body(d_ref, s_ref, b_ref, w_ref, g_ref):
    dis = lax.rsqrt(d_ref[...] + 1.0)
    sfull = jnp.concatenate([s_ref[0], s_ref[1]], axis=1)
    h = jnp.maximum(dis * sfull + b_ref[...], 0.0)
    m = jnp.dot(h, w_ref[...], preferred_element_type=jnp.float32)
    g_ref[0] = dis * m
    g_ref[1] = jnp.zeros((BR, 128), jnp.float32)


def _tc3_call(dcol, s3d, b, W3):
    return pl.pallas_call(
        _tc3_body,
        grid=(NBR,),
        in_specs=[
            pl.BlockSpec((BR, 1), lambda r: (r, 0)),
            pl.BlockSpec((2, BR, 128), lambda r: (0, r, 0)),
            pl.BlockSpec((1, 256), lambda r: (0, 0)),
            pl.BlockSpec((256, 128), lambda r: (0, 0)),
        ],
        out_specs=pl.BlockSpec((2, BR, 128), lambda r: (0, r, 0)),
        out_shape=jax.ShapeDtypeStruct((2, N, 128), jnp.float32),
    )(dcol, s3d, b, W3)


def _tc4_body(d_ref, s_ref, b_ref, o_ref):
    dis = lax.rsqrt(d_ref[...] + 1.0)
    o_ref[...] = dis * (s_ref[0] + s_ref[1]) + b_ref[...]


def _tc4_call(dcol, s3d, b):
    return pl.pallas_call(
        _tc4_body,
        grid=(NBR,),
        in_specs=[
            pl.BlockSpec((BR, 1), lambda r: (r, 0)),
            pl.BlockSpec((2, BR, 128), lambda r: (0, r, 0)),
            pl.BlockSpec((1, 128), lambda r: (0, 0)),
        ],
        out_specs=pl.BlockSpec((BR, 128), lambda r: (r, 0)),
        out_shape=jax.ShapeDtypeStruct((N, 128), jnp.float32),
    )(dcol, s3d, b)


def kernel(x, edge_index, W1, b1, W2, b2, W3, b3):
    edges_a = edge_index.reshape(2, NS, SLABS, E // NS // K // SLABS, K)
    edges_b = edge_index.reshape(2, NC * NS, SLABS,
                                 E // NC // NS // K // SLABS, K)
    deg = _deg_call(edges_a)
    dcol = deg.reshape(N, 1)
    g1 = _tc1_call(dcol, x, W1)
    s1 = _prop_call(g1.reshape(2 * N, 128), edges_a, edge_split=False)
    g2 = _tcmid_call(dcol, s1.reshape(2, N, 128), b1.reshape(1, 256), W2)
    s2 = _prop_call(g2.reshape(2 * N, 128), edges_a, edge_split=False)
    g3 = _tc3_call(dcol, s2.reshape(2, N, 128), b2.reshape(1, 256), W3)
    s3 = _prop_call(g3.reshape(2 * N, 128), edges_b, edge_split=True)
    out = _tc4_call(dcol, s3.reshape(2, N, 128), b3.reshape(1, 128))
    return out


# speedup vs baseline: 23.4379x; 1.8181x over previous
import functools

import jax
import jax.numpy as jnp
from jax import lax
from jax.experimental import pallas as pl
from jax.experimental.pallas import tpu as pltpu
from jax.experimental.pallas import tpu_sc as plsc

N = 10000
E = 320000
NC = 2
NS = 16
K = 80
NCS = 25
NB = 3
RT = 624
RT_LAST = N - 15 * RT
BR = 1000
NBR = N // BR

_MESH = dict(core_axis_name="c", subcore_axis_name="s")


def _tile_rows(s, copy_fn):
    @pl.when(s < 15)
    def _():
        copy_fn(s * RT, RT)

    @pl.when(s == 15)
    def _():
        copy_fn(15 * RT, RT_LAST)


def _deg_body(edges4d, deg_out, dstbig, ones_v, zb, deg_sh):
    c = lax.axis_index("c")
    s = lax.axis_index("s")
    zero16 = jnp.zeros((16,), jnp.float32)
    for j in range(40):
        zb[pl.ds(16 * j, 16)] = zero16
    one16 = jnp.full((16,), 1.0, jnp.float32)
    for j in range(K // 16):
        ones_v[pl.ds(16 * j, 16)] = one16
    @pl.when(s < 15)
    def _():
        pltpu.sync_copy(zb, deg_sh.at[pl.ds(s * 640, 640)])

    @pl.when(s == 15)
    def _():
        pltpu.sync_copy(zb.at[pl.ds(0, 400)], deg_sh.at[pl.ds(9600, 400)])

    plsc.subcore_barrier()

    @pl.when(c == 0)
    def _():
        nslabs = E // NS // K // NCS

        def slab(r, carry):
            pltpu.sync_copy(edges4d.at[1, s, r], dstbig)

            def it(i, c2):
                pltpu.sync_copy(ones_v, deg_sh.at[dstbig.at[i]], add=True)
                return c2

            return lax.fori_loop(0, NCS, it, carry)

        lax.fori_loop(0, nslabs, slab, 0)

    plsc.subcore_barrier()

    @pl.when(c == 0)
    def _():
        @pl.when(s < 15)
        def _():
            pltpu.sync_copy(deg_sh.at[pl.ds(s * 640, 640)], zb)
            pltpu.sync_copy(zb, deg_out.at[pl.ds(s * 640, 640)])

        @pl.when(s == 15)
        def _():
            pltpu.sync_copy(deg_sh.at[pl.ds(9600, 400)], zb.at[pl.ds(0, 400)])
            pltpu.sync_copy(zb.at[pl.ds(0, 400)], deg_out.at[pl.ds(9600, 400)])


def _deg_call(edges4d):
    f = functools.partial(
        pl.kernel,
        out_type=jax.ShapeDtypeStruct((N,), jnp.float32),
        mesh=plsc.VectorSubcoreMesh(**_MESH),
        scratch_types=[
            pltpu.VMEM((NCS, K), jnp.int32),
            pltpu.VMEM((K,), jnp.float32),
            pltpu.VMEM((640,), jnp.float32),
            pltpu.VMEM_SHARED((N,), jnp.float32),
        ],
    )(_deg_body)
    return f(edges4d)


def _make_prop_body(edge_split):
    nslabs = (E // NC if edge_split else E) // NS // K // NCS

    def body(g_hbm, edges4d, out_hbm, srcbig, dstbig, rows3, gsem, ssem,
             acc_sh):
        c = lax.axis_index("c")
        s = lax.axis_index("s")
        _tile_rows(s, lambda rb, nr: pltpu.sync_copy(
            g_hbm.at[pl.ds(c * N + rb, nr)], acc_sh.at[pl.ds(rb, nr)]))
        t = c * NS + s if edge_split else s
        plsc.subcore_barrier()

        def start_gather(l, b):
            pltpu.make_async_copy(
                g_hbm.at[srcbig.at[l]], rows3.at[b], gsem.at[b]).start()

        def wait_gather(l, b):
            pltpu.make_async_copy(
                g_hbm.at[srcbig.at[l]], rows3.at[b], gsem.at[b]).wait()

        def start_scatter(l, b):
            pltpu.make_async_copy(
                rows3.at[b], acc_sh.at[dstbig.at[l]], ssem.at[b]
            ).start(add=True)

        def wait_scatter(l, b):
            pltpu.make_async_copy(
                rows3.at[b], acc_sh.at[dstbig.at[l]], ssem.at[b]).wait()

        def slab(r, carry):
            @pl.when(r >= 1)
            def _():
                for b in range(NB):
                    wait_scatter(0, b)

            pltpu.sync_copy(edges4d.at[0, t, r], srcbig)
            pltpu.sync_copy(edges4d.at[1, t, r], dstbig)
            if not edge_split:
                bias = c * N

                def pre(i, c2):
                    for j in range(K // 16):
                        srcbig[i, pl.ds(16 * j, 16)] = (
                            srcbig[i, pl.ds(16 * j, 16)] + bias)
                    return c2

                lax.fori_loop(0, NCS, pre, 0)
            gbase = r * NCS
            for j in range(2):
                start_gather(j, lax.rem(gbase + j, NB))

            def chunk(l, c2):
                g = gbase + l
                b = lax.rem(g, NB)

                @pl.when(l + 2 < NCS)
                def _():
                    b2 = lax.rem(g + 2, NB)

                    @pl.when(l >= 1)
                    def _():
                        wait_scatter(l, b2)

                    start_gather(l + 2, b2)

                wait_gather(l, b)
                start_scatter(l, b)
                return c2

            return lax.fori_loop(0, NCS, chunk, carry)

        lax.fori_loop(0, nslabs, slab, 0)
        for b in range(NB):
            wait_scatter(0, b)
        plsc.subcore_barrier()
        _tile_rows(s, lambda rb, nr: pltpu.sync_copy(
            acc_sh.at[pl.ds(rb, nr)], out_hbm.at[pl.ds(c * N + rb, nr)]))

    return body


def _prop_call(g_flat, edges4d, edge_split):
    f = functools.partial(
        pl.kernel,
        out_type=jax.ShapeDtypeStruct((2 * N, 128), jnp.float32),
        mesh=plsc.VectorSubcoreMesh(**_MESH),
        scratch_types=[
            pltpu.VMEM((NCS, K), jnp.int32),
            pltpu.VMEM((NCS, K), jnp.int32),
            pltpu.VMEM((NB, K, 128), jnp.float32),
            pltpu.SemaphoreType.DMA((NB,)),
            pltpu.SemaphoreType.DMA((NB,)),
            pltpu.VMEM_SHARED((N, 128), jnp.float32),
        ],
    )(_make_prop_body(edge_split))
    return f(g_flat, edges4d)


def _tc1_body(d_ref, x_ref, w_ref, g_ref):
    dis = lax.rsqrt(d_ref[...] + 1.0)
    m = jnp.dot(x_ref[...], w_ref[...],
                preferred_element_type=jnp.float32)
    g = dis * m
    g_ref[0] = g[:, :128]
    g_ref[1] = g[:, 128:]


def _tc1_call(dcol, x, W1):
    return pl.pallas_call(
        _tc1_body,
        grid=(NBR,),
        in_specs=[
            pl.BlockSpec((BR, 1), lambda r: (r, 0)),
            pl.BlockSpec((BR, 128), lambda r: (r, 0)),
            pl.BlockSpec((128, 256), lambda r: (0, 0)),
        ],
        out_specs=pl.BlockSpec((2, BR, 128), lambda r: (0, r, 0)),
        out_shape=jax.ShapeDtypeStruct((2, N, 128), jnp.float32),
    )(dcol, x, W1)


def _tcmid_body(d_ref, s_ref, b_ref, w_ref, g_ref):
    dis = lax.rsqrt(d_ref[...] + 1.0)
    sfull = jnp.concatenate([s_ref[0], s_ref[1]], axis=1)
    h = jnp.maximum(dis * sfull + b_ref[...], 0.0)
    m = jnp.dot(h, w_ref[...], preferred_element_type=jnp.float32)
    g = dis * m
    g_ref[0] = g[:, :128]
    g_ref[1] = g[:, 128:]


def _tcmid_call(dcol, s3d, b, W):
    return pl.pallas_call(
        _tcmid_body,
        grid=(NBR,),
        in_specs=[
            pl.BlockSpec((BR, 1), lambda r: (r, 0)),
            pl.BlockSpec((2, BR, 128), lambda r: (0, r, 0)),
            pl.BlockSpec((1, 256), lambda r: (0, 0)),
            pl.BlockSpec((256, 256), lambda r: (0, 0)),
        ],
        out_specs=pl.BlockSpec((2, BR, 128), lambda r: (0, r, 0)),
        out_shape=jax.ShapeDtypeStruct((2, N, 128), jnp.float32),
    )(dcol, s3d, b, W)


def _tc3_body(d_ref, s_ref, b_ref, w_ref, g_ref):
    dis = lax.rsqrt(d_ref[...] + 1.0)
    sfull = jnp.concatenate([s_ref[0], s_ref[1]], axis=1)
    h = jnp.maximum(dis * sfull + b_ref[...], 0.0)
    m = jnp.dot(h, w_ref[...], preferred_element_type=jnp.float32)
    g_ref[0] = dis * m
    g_ref[1] = jnp.zeros((BR, 128), jnp.float32)


def _tc3_call(dcol, s3d, b, W3):
    return pl.pallas_call(
        _tc3_body,
        grid=(NBR,),
        in_specs=[
            pl.BlockSpec((BR, 1), lambda r: (r, 0)),
            pl.BlockSpec((2, BR, 128), lambda r: (0, r, 0)),
            pl.BlockSpec((1, 256), lambda r: (0, 0)),
            pl.BlockSpec((256, 128), lambda r: (0, 0)),
        ],
        out_specs=pl.BlockSpec((2, BR, 128), lambda r: (0, r, 0)),
        out_shape=jax.ShapeDtypeStruct((2, N, 128), jnp.float32),
    )(dcol, s3d, b, W3)


def _tc4_body(d_ref, s_ref, b_ref, o_ref):
    dis = lax.rsqrt(d_ref[...] + 1.0)
    o_ref[...] = dis * (s_ref[0] + s_ref[1]) + b_ref[...]


def _tc4_call(dcol, s3d, b):
    return pl.pallas_call(
        _tc4_body,
        grid=(NBR,),
        in_specs=[
            pl.BlockSpec((BR, 1), lambda r: (r, 0)),
            pl.BlockSpec((2, BR, 128), lambda r: (0, r, 0)),
            pl.BlockSpec((1, 128), lambda r: (0, 0)),
        ],
        out_specs=pl.BlockSpec((BR, 128), lambda r: (r, 0)),
        out_shape=jax.ShapeDtypeStruct((N, 128), jnp.float32),
    )(dcol, s3d, b)


def kernel(x, edge_index, W1, b1, W2, b2, W3, b3):
    edges_a = edge_index.reshape(2, NS, E // NS // K // NCS, NCS, K)
    edges_b = edge_index.reshape(2, NC * NS, E // NC // NS // K // NCS,
                                 NCS, K)
    deg = _deg_call(edges_a)
    dcol = deg.reshape(N, 1)
    g1 = _tc1_call(dcol, x, W1)
    s1 = _prop_call(g1.reshape(2 * N, 128), edges_a, edge_split=False)
    g2 = _tcmid_call(dcol, s1.reshape(2, N, 128), b1.reshape(1, 256), W2)
    s2 = _prop_call(g2.reshape(2 * N, 128), edges_a, edge_split=False)
    g3 = _tc3_call(dcol, s2.reshape(2, N, 128), b2.reshape(1, 256), W3)
    s3 = _prop_call(g3.reshape(2 * N, 128), edges_b, edge_split=True)
    out = _tc4_call(dcol, s3.reshape(2, N, 128), b3.reshape(1, 128))
    return out


# speedup vs baseline: 25.0849x; 1.0703x over previous
import functools

import jax
import jax.numpy as jnp
from jax import lax
from jax.experimental import pallas as pl
from jax.experimental.pallas import tpu as pltpu
from jax.experimental.pallas import tpu_sc as plsc

N = 10000
E = 320000
NC = 2
NS = 16
K = 80
NCS = 25
NB = 3
RT = 624
RT_LAST = N - 15 * RT
BR = 1000
NBR = N // BR

_MESH = dict(core_axis_name="c", subcore_axis_name="s")


def _tile_rows(s, copy_fn):
    @pl.when(s < 15)
    def _():
        copy_fn(s * RT, RT)

    @pl.when(s == 15)
    def _():
        copy_fn(15 * RT, RT_LAST)


def _deg_body(edges4d, deg0_out, deg1_out, dstbig, ones_v, zb, dsem, deg_sh):
    c = lax.axis_index("c")
    s = lax.axis_index("s")
    zero16 = jnp.zeros((16,), jnp.float32)
    for j in range(40):
        zb[pl.ds(16 * j, 16)] = zero16
    one16 = jnp.full((16,), 1.0, jnp.float32)
    for j in range(K // 16):
        ones_v[pl.ds(16 * j, 16)] = one16
    @pl.when(s < 15)
    def _():
        pltpu.sync_copy(zb, deg_sh.at[pl.ds(s * 640, 640)])

    @pl.when(s == 15)
    def _():
        pltpu.sync_copy(zb.at[pl.ds(0, 400)], deg_sh.at[pl.ds(9600, 400)])

    plsc.subcore_barrier()

    t = c * NS + s
    nslabs = E // NC // NS // K // NCS

    def slab(r, carry):
        pltpu.sync_copy(edges4d.at[1, t, r], dstbig)

        def fire(i, c2):
            pltpu.make_async_copy(
                ones_v, deg_sh.at[dstbig.at[i]], dsem).start(add=True)
            return c2

        lax.fori_loop(0, NCS, fire, 0)

        def drain(i, c2):
            pltpu.make_async_copy(
                ones_v, deg_sh.at[dstbig.at[i]], dsem).wait()
            return c2

        return lax.fori_loop(0, NCS, drain, carry)

    lax.fori_loop(0, nslabs, slab, 0)

    plsc.subcore_barrier()

    def wb(dego):
        @pl.when(s < 15)
        def _():
            pltpu.sync_copy(deg_sh.at[pl.ds(s * 640, 640)], zb)
            pltpu.sync_copy(zb, dego.at[pl.ds(s * 640, 640)])

        @pl.when(s == 15)
        def _():
            pltpu.sync_copy(deg_sh.at[pl.ds(9600, 400)], zb.at[pl.ds(0, 400)])
            pltpu.sync_copy(zb.at[pl.ds(0, 400)], dego.at[pl.ds(9600, 400)])

    @pl.when(c == 0)
    def _():
        wb(deg0_out)

    @pl.when(c == 1)
    def _():
        wb(deg1_out)


def _deg_call(edges4d):
    f = functools.partial(
        pl.kernel,
        out_type=(jax.ShapeDtypeStruct((N,), jnp.float32),
                  jax.ShapeDtypeStruct((N,), jnp.float32)),
        mesh=plsc.VectorSubcoreMesh(**_MESH),
        scratch_types=[
            pltpu.VMEM((NCS, K), jnp.int32),
            pltpu.VMEM((K,), jnp.float32),
            pltpu.VMEM((640,), jnp.float32),
            pltpu.SemaphoreType.DMA,
            pltpu.VMEM_SHARED((N,), jnp.float32),
        ],
    )(_deg_body)
    return f(edges4d)


def _make_prop_body(edge_split):
    nslabs = (E // NC if edge_split else E) // NS // K // NCS

    def body(g_hbm, edges4d, out_hbm, srcbig, dstbig, rows3, gsem, ssem,
             isem, acc_sh):
        c = lax.axis_index("c")
        s = lax.axis_index("s")
        _tile_rows(s, lambda rb, nr: pltpu.sync_copy(
            g_hbm.at[pl.ds(c * N + rb, nr)], acc_sh.at[pl.ds(rb, nr)]))
        t = c * NS + s if edge_split else s
        gtab = g_hbm if edge_split else g_hbm.at[pl.ds(c * N, N)]
        plsc.subcore_barrier()

        def start_gather(r2, l, b):
            pltpu.make_async_copy(
                gtab.at[srcbig.at[r2, l]], rows3.at[b], gsem.at[b]).start()

        def wait_gather(r2, l, b):
            pltpu.make_async_copy(
                gtab.at[srcbig.at[r2, l]], rows3.at[b], gsem.at[b]).wait()

        def start_scatter(r2, l, b):
            pltpu.make_async_copy(
                rows3.at[b], acc_sh.at[dstbig.at[r2, l]], ssem.at[b]
            ).start(add=True)

        def wait_scatter(b):
            pltpu.make_async_copy(
                rows3.at[b], acc_sh.at[dstbig.at[0, 0]], ssem.at[b]).wait()

        def fetch_slab(r, r2):
            pltpu.make_async_copy(
                edges4d.at[0, t, r], srcbig.at[r2], isem).start()
            pltpu.make_async_copy(
                edges4d.at[1, t, r], dstbig.at[r2], isem).start()

        def wait_slab(r2):
            pltpu.make_async_copy(
                edges4d.at[0, t, 0], srcbig.at[r2], isem).wait()
            pltpu.make_async_copy(
                edges4d.at[1, t, 0], dstbig.at[r2], isem).wait()

        fetch_slab(0, 0)
        la = NB - 1

        def slab(r, carry):
            @pl.when(r >= 1)
            def _():
                for b in range(NB):
                    wait_scatter(b)

            r2 = lax.rem(r, 2)
            wait_slab(r2)

            @pl.when(r + 1 < nslabs)
            def _():
                fetch_slab(r + 1, lax.rem(r + 1, 2))

            gbase = r * NCS
            for j in range(la):
                start_gather(r2, j, lax.rem(gbase + j, NB))

            def chunk(l, c2):
                g = gbase + l
                b = lax.rem(g, NB)

                @pl.when(l + la < NCS)
                def _():
                    b2 = lax.rem(g + la, NB)

                    @pl.when(l >= 1)
                    def _():
                        wait_scatter(b2)

                    start_gather(r2, l + la, b2)

                wait_gather(r2, l, b)
                start_scatter(r2, l, b)
                return c2

            return lax.fori_loop(0, NCS, chunk, carry)

        lax.fori_loop(0, nslabs, slab, 0)
        for b in range(NB):
            wait_scatter(b)
        plsc.subcore_barrier()
        _tile_rows(s, lambda rb, nr: pltpu.sync_copy(
            acc_sh.at[pl.ds(rb, nr)], out_hbm.at[pl.ds(c * N + rb, nr)]))

    return body


def _prop_call(g_flat, edges4d, edge_split):
    f = functools.partial(
        pl.kernel,
        out_type=jax.ShapeDtypeStruct((2 * N, 128), jnp.float32),
        mesh=plsc.VectorSubcoreMesh(**_MESH),
        scratch_types=[
            pltpu.VMEM((2, NCS, K), jnp.int32),
            pltpu.VMEM((2, NCS, K), jnp.int32),
            pltpu.VMEM((NB, K, 128), jnp.float32),
            pltpu.SemaphoreType.DMA((NB,)),
            pltpu.SemaphoreType.DMA((NB,)),
            pltpu.SemaphoreType.DMA,
            pltpu.VMEM_SHARED((N, 128), jnp.float32),
        ],
    )(_make_prop_body(edge_split))
    return f(g_flat, edges4d)


def _tc1_body(d0_ref, d1_ref, x_ref, w_ref, g_ref):
    dis = lax.rsqrt(d0_ref[...] + d1_ref[...] + 1.0)
    m = jnp.dot(x_ref[...], w_ref[...],
                preferred_element_type=jnp.float32)
    g = dis * m
    g_ref[0] = g[:, :128]
    g_ref[1] = g[:, 128:]


def _tc1_call(dcol0, dcol1, x, W1):
    return pl.pallas_call(
        _tc1_body,
        grid=(NBR,),
        in_specs=[
            pl.BlockSpec((BR, 1), lambda r: (r, 0)),
            pl.BlockSpec((BR, 1), lambda r: (r, 0)),
            pl.BlockSpec((BR, 128), lambda r: (r, 0)),
            pl.BlockSpec((128, 256), lambda r: (0, 0)),
        ],
        out_specs=pl.BlockSpec((2, BR, 128), lambda r: (0, r, 0)),
        out_shape=jax.ShapeDtypeStruct((2, N, 128), jnp.float32),
    )(dcol0, dcol1, x, W1)


def _tcmid_body(d0_ref, d1_ref, s_ref, b_ref, w_ref, g_ref):
    dis = lax.rsqrt(d0_ref[...] + d1_ref[...] + 1.0)
    sfull = jnp.concatenate([s_ref[0], s_ref[1]], axis=1)
    h = jnp.maximum(dis * sfull + b_ref[...], 0.0)
    m = jnp.dot(h, w_ref[...], preferred_element_type=jnp.float32)
    g = dis * m
    g_ref[0] = g[:, :128]
    g_ref[1] = g[:, 128:]


def _tcmid_call(dcol0, dcol1, s3d, b, W):
    return pl.pallas_call(
        _tcmid_body,
        grid=(NBR,),
        in_specs=[
            pl.BlockSpec((BR, 1), lambda r: (r, 0)),
            pl.BlockSpec((BR, 1), lambda r: (r, 0)),
            pl.BlockSpec((2, BR, 128), lambda r: (0, r, 0)),
            pl.BlockSpec((1, 256), lambda r: (0, 0)),
            pl.BlockSpec((256, 256), lambda r: (0, 0)),
        ],
        out_specs=pl.BlockSpec((2, BR, 128), lambda r: (0, r, 0)),
        out_shape=jax.ShapeDtypeStruct((2, N, 128), jnp.float32),
    )(dcol0, dcol1, s3d, b, W)


def _tc3_body(d0_ref, d1_ref, s_ref, b_ref, w_ref, g_ref):
    dis = lax.rsqrt(d0_ref[...] + d1_ref[...] + 1.0)
    sfull = jnp.concatenate([s_ref[0], s_ref[1]], axis=1)
    h = jnp.maximum(dis * sfull + b_ref[...], 0.0)
    m = jnp.dot(h, w_ref[...], preferred_element_type=jnp.float32)
    g_ref[0] = dis * m
    g_ref[1] = jnp.zeros((BR, 128), jnp.float32)


def _tc3_call(dcol0, dcol1, s3d, b, W3):
    return pl.pallas_call(
        _tc3_body,
        grid=(NBR,),
        in_specs=[
            pl.BlockSpec((BR, 1), lambda r: (r, 0)),
            pl.BlockSpec((BR, 1), lambda r: (r, 0)),
            pl.BlockSpec((2, BR, 128), lambda r: (0, r, 0)),
            pl.BlockSpec((1, 256), lambda r: (0, 0)),
            pl.BlockSpec((256, 128), lambda r: (0, 0)),
        ],
        out_specs=pl.BlockSpec((2, BR, 128), lambda r: (0, r, 0)),
        out_shape=jax.ShapeDtypeStruct((2, N, 128), jnp.float32),
    )(dcol0, dcol1, s3d, b, W3)


def _tc4_body(d0_ref, d1_ref, s_ref, b_ref, o_ref):
    dis = lax.rsqrt(d0_ref[...] + d1_ref[...] + 1.0)
    o_ref[...] = dis * (s_ref[0] + s_ref[1]) + b_ref[...]


def _tc4_call(dcol0, dcol1, s3d, b):
    return pl.pallas_call(
        _tc4_body,
        grid=(NBR,),
        in_specs=[
            pl.BlockSpec((BR, 1), lambda r: (r, 0)),
            pl.BlockSpec((BR, 1), lambda r: (r, 0)),
            pl.BlockSpec((2, BR, 128), lambda r: (0, r, 0)),
            pl.BlockSpec((1, 128), lambda r: (0, 0)),
        ],
        out_specs=pl.BlockSpec((BR, 128), lambda r: (r, 0)),
        out_shape=jax.ShapeDtypeStruct((N, 128), jnp.float32),
    )(dcol0, dcol1, s3d, b)


def kernel(x, edge_index, W1, b1, W2, b2, W3, b3):
    edges_a = edge_index.reshape(2, NS, E // NS // K // NCS, NCS, K)
    edges_b = edge_index.reshape(2, NC * NS, E // NC // NS // K // NCS,
                                 NCS, K)
    deg0, deg1 = _deg_call(edges_b)
    dcol0 = deg0.reshape(N, 1)
    dcol1 = deg1.reshape(N, 1)
    g1 = _tc1_call(dcol0, dcol1, x, W1)
    s1 = _prop_call(g1.reshape(2 * N, 128), edges_a, edge_split=False)
    g2 = _tcmid_call(dcol0, dcol1, s1.reshape(2, N, 128), b1.reshape(1, 256), W2)
    s2 = _prop_call(g2.reshape(2 * N, 128), edges_a, edge_split=False)
    g3 = _tc3_call(dcol0, dcol1, s2.reshape(2, N, 128), b2.reshape(1, 256), W3)
    s3 = _prop_call(g3.reshape(2 * N, 128), edges_b, edge_split=True)
    out = _tc4_call(dcol0, dcol1, s3.reshape(2, N, 128), b3.reshape(1, 128))
    return out


# speedup vs baseline: 25.2053x; 1.0048x over previous
import functools

import jax
import jax.numpy as jnp
from jax import lax
from jax.experimental import pallas as pl
from jax.experimental.pallas import tpu as pltpu
from jax.experimental.pallas import tpu_sc as plsc

N = 10000
E = 320000
NC = 2
NS = 16
K = 80
NCS = 25
NB = 3
RT = 624
RT_LAST = N - 15 * RT
BR = 1000
NBR = N // BR

_MESH = dict(core_axis_name="c", subcore_axis_name="s")


def _tile_rows(s, copy_fn):
    @pl.when(s < 15)
    def _():
        copy_fn(s * RT, RT)

    @pl.when(s == 15)
    def _():
        copy_fn(15 * RT, RT_LAST)


def _deg_body(edges4d, deg0_out, deg1_out, dstbig, ones_v, zb, dsem, deg_sh):
    c = lax.axis_index("c")
    s = lax.axis_index("s")
    zero16 = jnp.zeros((16,), jnp.float32)
    for j in range(40):
        zb[pl.ds(16 * j, 16)] = zero16
    one16 = jnp.full((16,), 1.0, jnp.float32)
    for j in range(K // 16):
        ones_v[pl.ds(16 * j, 16)] = one16
    @pl.when(s < 15)
    def _():
        pltpu.sync_copy(zb, deg_sh.at[pl.ds(s * 640, 640)])

    @pl.when(s == 15)
    def _():
        pltpu.sync_copy(zb.at[pl.ds(0, 400)], deg_sh.at[pl.ds(9600, 400)])

    plsc.subcore_barrier()

    t = c * NS + s
    nslabs = E // NC // NS // K // NCS

    def slab(r, carry):
        pltpu.sync_copy(edges4d.at[1, t, r], dstbig)

        def fire(i, c2):
            pltpu.make_async_copy(
                ones_v, deg_sh.at[dstbig.at[i]], dsem).start(add=True)
            return c2

        lax.fori_loop(0, NCS, fire, 0)

        def drain(i, c2):
            pltpu.make_async_copy(
                ones_v, deg_sh.at[dstbig.at[i]], dsem).wait()
            return c2

        return lax.fori_loop(0, NCS, drain, carry)

    lax.fori_loop(0, nslabs, slab, 0)

    plsc.subcore_barrier()

    def wb(dego):
        @pl.when(s < 15)
        def _():
            pltpu.sync_copy(deg_sh.at[pl.ds(s * 640, 640)], zb)
            pltpu.sync_copy(zb, dego.at[pl.ds(s * 640, 640)])

        @pl.when(s == 15)
        def _():
            pltpu.sync_copy(deg_sh.at[pl.ds(9600, 400)], zb.at[pl.ds(0, 400)])
            pltpu.sync_copy(zb.at[pl.ds(0, 400)], dego.at[pl.ds(9600, 400)])

    @pl.when(c == 0)
    def _():
        wb(deg0_out)

    @pl.when(c == 1)
    def _():
        wb(deg1_out)


def _deg_call(edges4d):
    f = functools.partial(
        pl.kernel,
        out_type=(jax.ShapeDtypeStruct((N,), jnp.float32),
                  jax.ShapeDtypeStruct((N,), jnp.float32)),
        mesh=plsc.VectorSubcoreMesh(**_MESH),
        scratch_types=[
            pltpu.VMEM((NCS, K), jnp.int32),
            pltpu.VMEM((K,), jnp.float32),
            pltpu.VMEM((640,), jnp.float32),
            pltpu.SemaphoreType.DMA,
            pltpu.VMEM_SHARED((N,), jnp.float32),
        ],
    )(_deg_body)
    return f(edges4d)


def _make_prop_body(edge_split):
    nslabs = (E // NC if edge_split else E) // NS // K // NCS

    def body(g_hbm, edges4d, out_hbm, srcbig, dstbig, rows3, gsem, ssem,
             isem, acc_sh):
        c = lax.axis_index("c")
        s = lax.axis_index("s")
        t = c * NS + s if edge_split else s
        gtab = g_hbm if edge_split else g_hbm.at[pl.ds(c * N, N)]

        def start_gather(r2, l, b):
            pltpu.make_async_copy(
                gtab.at[srcbig.at[r2, l]], rows3.at[b], gsem.at[b]).start()

        def wait_gather(r2, l, b):
            pltpu.make_async_copy(
                gtab.at[srcbig.at[r2, l]], rows3.at[b], gsem.at[b]).wait()

        def start_scatter(r2, l, b):
            pltpu.make_async_copy(
                rows3.at[b], acc_sh.at[dstbig.at[r2, l]], ssem.at[b]
            ).start(add=True)

        def wait_scatter(b):
            pltpu.make_async_copy(
                rows3.at[b], acc_sh.at[dstbig.at[0, 0]], ssem.at[b]).wait()

        def fetch_slab(r, r2):
            pltpu.make_async_copy(
                edges4d.at[0, t, r], srcbig.at[r2], isem).start()
            pltpu.make_async_copy(
                edges4d.at[1, t, r], dstbig.at[r2], isem).start()

        def wait_slab(r2):
            pltpu.make_async_copy(
                edges4d.at[0, t, 0], srcbig.at[r2], isem).wait()
            pltpu.make_async_copy(
                edges4d.at[1, t, 0], dstbig.at[r2], isem).wait()

        fetch_slab(0, 0)
        _tile_rows(s, lambda rb, nr: pltpu.sync_copy(
            g_hbm.at[pl.ds(c * N + rb, nr)], acc_sh.at[pl.ds(rb, nr)]))
        plsc.subcore_barrier()
        la = NB - 1

        def slab(r, carry):
            @pl.when(r >= 1)
            def _():
                for b in range(NB):
                    wait_scatter(b)

            r2 = lax.rem(r, 2)
            wait_slab(r2)

            @pl.when(r + 1 < nslabs)
            def _():
                fetch_slab(r + 1, lax.rem(r + 1, 2))

            gbase = r * NCS
            for j in range(la):
                start_gather(r2, j, lax.rem(gbase + j, NB))

            def chunk(l, c2):
                g = gbase + l
                b = lax.rem(g, NB)

                @pl.when(l + la < NCS)
                def _():
                    b2 = lax.rem(g + la, NB)

                    @pl.when(l >= 1)
                    def _():
                        wait_scatter(b2)

                    start_gather(r2, l + la, b2)

                wait_gather(r2, l, b)
                start_scatter(r2, l, b)
                return c2

            return lax.fori_loop(0, NCS, chunk, carry)

        lax.fori_loop(0, nslabs, slab, 0)
        for b in range(NB):
            wait_scatter(b)
        plsc.subcore_barrier()
        _tile_rows(s, lambda rb, nr: pltpu.sync_copy(
            acc_sh.at[pl.ds(rb, nr)], out_hbm.at[pl.ds(c * N + rb, nr)]))

    return body


def _prop_call(g_flat, edges4d, edge_split):
    f = functools.partial(
        pl.kernel,
        out_type=jax.ShapeDtypeStruct((2 * N, 128), jnp.float32),
        mesh=plsc.VectorSubcoreMesh(**_MESH),
        scratch_types=[
            pltpu.VMEM((2, NCS, K), jnp.int32),
            pltpu.VMEM((2, NCS, K), jnp.int32),
            pltpu.VMEM((NB, K, 128), jnp.float32),
            pltpu.SemaphoreType.DMA((NB,)),
            pltpu.SemaphoreType.DMA((NB,)),
            pltpu.SemaphoreType.DMA,
            pltpu.VMEM_SHARED((N, 128), jnp.float32),
        ],
    )(_make_prop_body(edge_split))
    return f(g_flat, edges4d)


def _tc1_body(d0_ref, d1_ref, x_ref, w_ref, g_ref):
    dis = lax.rsqrt(d0_ref[...] + d1_ref[...] + 1.0)
    m = jnp.dot(x_ref[...], w_ref[...],
                preferred_element_type=jnp.float32)
    g = dis * m
    g_ref[0] = g[:, :128]
    g_ref[1] = g[:, 128:]


def _tc1_call(dcol0, dcol1, x, W1):
    return pl.pallas_call(
        _tc1_body,
        grid=(NBR,),
        in_specs=[
            pl.BlockSpec((BR, 1), lambda r: (r, 0)),
            pl.BlockSpec((BR, 1), lambda r: (r, 0)),
            pl.BlockSpec((BR, 128), lambda r: (r, 0)),
            pl.BlockSpec((128, 256), lambda r: (0, 0)),
        ],
        out_specs=pl.BlockSpec((2, BR, 128), lambda r: (0, r, 0)),
        out_shape=jax.ShapeDtypeStruct((2, N, 128), jnp.float32),
    )(dcol0, dcol1, x, W1)


def _tcmid_body(d0_ref, d1_ref, s_ref, b_ref, w_ref, g_ref):
    dis = lax.rsqrt(d0_ref[...] + d1_ref[...] + 1.0)
    sfull = jnp.concatenate([s_ref[0], s_ref[1]], axis=1)
    h = jnp.maximum(dis * sfull + b_ref[...], 0.0)
    m = jnp.dot(h, w_ref[...], preferred_element_type=jnp.float32)
    g = dis * m
    g_ref[0] = g[:, :128]
    g_ref[1] = g[:, 128:]


def _tcmid_call(dcol0, dcol1, s3d, b, W):
    return pl.pallas_call(
        _tcmid_body,
        grid=(NBR,),
        in_specs=[
            pl.BlockSpec((BR, 1), lambda r: (r, 0)),
            pl.BlockSpec((BR, 1), lambda r: (r, 0)),
            pl.BlockSpec((2, BR, 128), lambda r: (0, r, 0)),
            pl.BlockSpec((1, 256), lambda r: (0, 0)),
            pl.BlockSpec((256, 256), lambda r: (0, 0)),
        ],
        out_specs=pl.BlockSpec((2, BR, 128), lambda r: (0, r, 0)),
        out_shape=jax.ShapeDtypeStruct((2, N, 128), jnp.float32),
    )(dcol0, dcol1, s3d, b, W)


def _tc3_body(d0_ref, d1_ref, s_ref, b_ref, w_ref, g_ref):
    dis = lax.rsqrt(d0_ref[...] + d1_ref[...] + 1.0)
    sfull = jnp.concatenate([s_ref[0], s_ref[1]], axis=1)
    h = jnp.maximum(dis * sfull + b_ref[...], 0.0)
    m = jnp.dot(h, w_ref[...], preferred_element_type=jnp.float32)
    g_ref[0] = dis * m
    g_ref[1] = jnp.zeros((BR, 128), jnp.float32)


def _tc3_call(dcol0, dcol1, s3d, b, W3):
    return pl.pallas_call(
        _tc3_body,
        grid=(NBR,),
        in_specs=[
            pl.BlockSpec((BR, 1), lambda r: (r, 0)),
            pl.BlockSpec((BR, 1), lambda r: (r, 0)),
            pl.BlockSpec((2, BR, 128), lambda r: (0, r, 0)),
            pl.BlockSpec((1, 256), lambda r: (0, 0)),
            pl.BlockSpec((256, 128), lambda r: (0, 0)),
        ],
        out_specs=pl.BlockSpec((2, BR, 128), lambda r: (0, r, 0)),
        out_shape=jax.ShapeDtypeStruct((2, N, 128), jnp.float32),
    )(dcol0, dcol1, s3d, b, W3)


def _tc4_body(d0_ref, d1_ref, s_ref, b_ref, o_ref):
    dis = lax.rsqrt(d0_ref[...] + d1_ref[...] + 1.0)
    o_ref[...] = dis * (s_ref[0] + s_ref[1]) + b_ref[...]


def _tc4_call(dcol0, dcol1, s3d, b):
    return pl.pallas_call(
        _tc4_body,
        grid=(NBR,),
        in_specs=[
            pl.BlockSpec((BR, 1), lambda r: (r, 0)),
            pl.BlockSpec((BR, 1), lambda r: (r, 0)),
            pl.BlockSpec((2, BR, 128), lambda r: (0, r, 0)),
            pl.BlockSpec((1, 128), lambda r: (0, 0)),
        ],
        out_specs=pl.BlockSpec((BR, 128), lambda r: (r, 0)),
        out_shape=jax.ShapeDtypeStruct((N, 128), jnp.float32),
    )(dcol0, dcol1, s3d, b)


def kernel(x, edge_index, W1, b1, W2, b2, W3, b3):
    edges_a = edge_index.reshape(2, NS, E // NS // K // NCS, NCS, K)
    edges_b = edge_index.reshape(2, NC * NS, E // NC // NS // K // NCS,
                                 NCS, K)
    deg0, deg1 = _deg_call(edges_b)
    dcol0 = deg0.reshape(N, 1)
    dcol1 = deg1.reshape(N, 1)
    g1 = _tc1_call(dcol0, dcol1, x, W1)
    s1 = _prop_call(g1.reshape(2 * N, 128), edges_a, edge_split=False)
    g2 = _tcmid_call(dcol0, dcol1, s1.reshape(2, N, 128), b1.reshape(1, 256), W2)
    s2 = _prop_call(g2.reshape(2 * N, 128), edges_a, edge_split=False)
    g3 = _tc3_call(dcol0, dcol1, s2.reshape(2, N, 128), b2.reshape(1, 256), W3)
    s3 = _prop_call(g3.reshape(2 * N, 128), edges_b, edge_split=True)
    out = _tc4_call(dcol0, dcol1, s3.reshape(2, N, 128), b3.reshape(1, 128))
    return out
